# baseline (speedup 1.0000x reference)
import dataclasses
import functools

import jax
import jax.numpy as jnp
from jax import lax
from jax.experimental import pallas as pl
from jax.experimental.pallas import tpu as pltpu
from jax.experimental.pallas import tpu_sc as plsc

N = 10000
E = 320000
D = 128
NUM_LAYERS = 2
NUM_CLUSTERS = 4
NUM_GRAPHS = 16

H0 = 5120
TH = 5248
CHUNK = 128
NWORK = 32
EW = E // NWORK
BLK = 2000
CAPC = 80
CAP = CAPC * CHUNK
NBKT = 2 * NUM_CLUSTERS
NREG = NBKT * NWORK
PSH = 8192

_mesh = plsc.VectorSubcoreMesh(core_axis_name="c", subcore_axis_name="s")

_cp = pltpu.CompilerParams()
if "needs_layout_passes" in pltpu.CompilerParams.__dataclass_fields__:
    _cp = dataclasses.replace(_cp, needs_layout_passes=False)


@functools.partial(
    pl.kernel,
    out_type=(
        jax.ShapeDtypeStruct((NREG * CAP,), jnp.int32),
        jax.ShapeDtypeStruct((NWORK, 16), jnp.int32),
    ),
    mesh=_mesh,
    scratch_types=[
        pltpu.VMEM((BLK,), jnp.int32),
        pltpu.VMEM((BLK,), jnp.int32),
        pltpu.VMEM((N,), jnp.int32),
        pltpu.VMEM((CAP,), jnp.int32),
        pltpu.VMEM((CAP,), jnp.int32),
        pltpu.VMEM((CAP,), jnp.int32),
        pltpu.VMEM((CAP,), jnp.int32),
        pltpu.VMEM((CAP,), jnp.int32),
        pltpu.VMEM((CAP,), jnp.int32),
        pltpu.VMEM((CAP,), jnp.int32),
        pltpu.VMEM((CAP,), jnp.int32),
        pltpu.VMEM((16,), jnp.int32),
    ],
    compiler_params=_cp,
)
def _sc_part(src_hbm, dst_hbm, lab_hbm, bp_hbm, cnt_hbm,
             sv, dv, lab, b0, b1, b2, b3, b4, b5, b6, b7, cv):
    cid = lax.axis_index("c")
    sid = lax.axis_index("s")
    w = sid * 2 + cid
    bb = (b0, b1, b2, b3, b4, b5, b6, b7)
    pltpu.sync_copy(lab_hbm, lab)
    lane = lax.iota(jnp.int32, 16)

    cnt = tuple(jnp.int32(0) for _ in range(NBKT))
    for blk in range(EW // BLK):
        pltpu.sync_copy(src_hbm.at[pl.ds(w * EW + blk * BLK, BLK)], sv)
        pltpu.sync_copy(dst_hbm.at[pl.ds(w * EW + blk * BLK, BLK)], dv)

        def body(v, c8):
            s16 = sv[pl.ds(v * 16, 16)]
            d16 = dv[pl.ds(v * 16, 16)]
            k16 = plsc.load_gather(lab, [d16])
            h16 = (d16 >= H0).astype(jnp.int32)
            b16 = k16 * 2 + h16
            p16 = s16 * PSH + (d16 - h16 * H0)
            out = []
            for b in range(NBKT):
                m = b16 == b
                plsc.store_compressed(bb[b].at[pl.ds(c8[b], 16)], p16, mask=m)
                nb_ = jnp.max(plsc.all_reduce_population_count(m))
                out.append(c8[b] + nb_)
            return tuple(out)

        cnt = lax.fori_loop(0, BLK // 16, body, cnt)

    cvec = jnp.zeros((16,), jnp.int32)
    for b in range(NBKT):
        cb = cnt[b]
        for t in range(8):
            off = cb + t * 16
            pad = lane + (t * 16 + w * 128)
            bb[b][pl.ds(off, 16)] = (pad % N) * PSH + H0 + pad % (TH - H0)
        nch = (cb + CHUNK) // CHUNK
        cvec = jnp.where(lane == b, jnp.broadcast_to(nch, (16,)), cvec)
        base = (b * NWORK + w) * CAP
        nblk = (cb + CHUNK + 1023) // 1024

        @pl.loop(0, nblk)
        def _(q):
            pltpu.sync_copy(bb[b].at[pl.ds(q * 1024, 1024)],
                            bp_hbm.at[pl.ds(base + q * 1024, 1024)])

    cv[...] = cvec
    pltpu.sync_copy(cv, cnt_hbm.at[w])


def _make_sc_agg(j):
    @functools.partial(
        pl.kernel,
        out_type=jax.ShapeDtypeStruct((2 * TH, D), jnp.float32),
        mesh=_mesh,
        scratch_types=[
            pltpu.VMEM((CAP // 2,), jnp.int32),
            pltpu.VMEM((4, CHUNK), jnp.int32),
            pltpu.VMEM((4, CHUNK), jnp.int32),
            pltpu.VMEM((CHUNK, D), jnp.float32),
            pltpu.VMEM((CHUNK, D), jnp.float32),
            pltpu.VMEM((CHUNK, D), jnp.float32),
            pltpu.VMEM((CHUNK, D), jnp.float32),
            pltpu.VMEM((CHUNK, D), jnp.float32),
            pltpu.VMEM((16,), jnp.int32),
            pltpu.VMEM((16,), jnp.int32),
            pltpu.VMEM_SHARED((TH, D), jnp.float32),
            pltpu.SemaphoreType.DMA,
            pltpu.SemaphoreType.DMA,
            pltpu.SemaphoreType.DMA,
            pltpu.SemaphoreType.DMA,
        ],
        compiler_params=_cp,
    )
    def _sc_agg(h_hbm, bp_hbm, cnt_hbm, out_hbm,
                pb, si2, di2, rows0, rows1, rows2, rows3, zbuf, cv0, cv1, table,
                sem0, sem1, sem2, sem3):
        cid = lax.axis_index("c")
        sid = lax.axis_index("s")
        HALF = CAPC // 2
        b = j * 2 + cid
        lane = lax.iota(jnp.int32, 16)
        zr = TH // 16

        pltpu.sync_copy(cnt_hbm.at[sid * 2], cv0)
        nch0 = jnp.max(jnp.where(lane == b, cv0[...], 0))
        cnt00 = jnp.clip(nch0, 0, HALF)
        base0 = (b * NWORK + sid * 2) * CAP

        def unpack(k, slot):
            for t in range(CHUNK // 16):
                pv = pb[pl.ds(k * CHUNK + t * 16, 16)]
                si2.at[slot][pl.ds(t * 16, 16)] = lax.shift_right_logical(
                    pv, jnp.int32(13))
                di2.at[slot][pl.ds(t * 16, 16)] = lax.bitwise_and(
                    pv, jnp.int32(PSH - 1))

        ROWS = (rows0, rows1, rows2, rows3)
        SEMS = (sem0, sem1, sem2, sem3)

        def prime(cnt_s):
            for u in range(4):
                @pl.when(cnt_s > u)
                def _(u=u):
                    unpack(u, u)
                    pltpu.async_copy(h_hbm.at[si2.at[u]], ROWS[u], SEMS[u])

        def drain(cnt_s):
            @pl.loop(0, cnt_s // 4)
            def _(kk):
                k0 = kk * 4
                for u in range(4):
                    pltpu.make_async_copy(h_hbm.at[si2.at[u]], ROWS[u],
                                          SEMS[u]).wait()
                    pltpu.sync_copy(ROWS[u], table.at[di2.at[u]], add=True)

                    @pl.when(k0 + u + 4 < cnt_s)
                    def _(u=u):
                        unpack(k0 + u + 4, u)
                        pltpu.async_copy(h_hbm.at[si2.at[u]], ROWS[u], SEMS[u])

            rem = cnt_s % 4
            for u in range(3):
                @pl.when(rem > u)
                def _(u=u):
                    pltpu.make_async_copy(h_hbm.at[si2.at[u]], ROWS[u],
                                          SEMS[u]).wait()
                    pltpu.sync_copy(ROWS[u], table.at[di2.at[u]], add=True)

        @pl.when(cnt00 > 0)
        def _():
            pltpu.sync_copy(bp_hbm.at[pl.ds(base0, CAP // 2)], pb)
            prime(cnt00)

        z16 = jnp.zeros((16,), jnp.float32)

        @pl.loop(0, CHUNK)
        def _(r):
            for t in range(D // 16):
                zbuf[r, pl.ds(t * 16, 16)] = z16

        pltpu.sync_copy(zbuf, table.at[pl.ds(sid * zr, CHUNK)])
        pltpu.sync_copy(zbuf, table.at[pl.ds(sid * zr + CHUNK, CHUNK)])
        pltpu.sync_copy(zbuf.at[pl.ds(0, zr - 2 * CHUNK)],
                        table.at[pl.ds(sid * zr + 2 * CHUNK, zr - 2 * CHUNK)])
        plsc.subcore_barrier()

        for r2 in range(2):
            r = sid * 2 + r2
            if r2 == 0:
                nch = nch0
            else:
                pltpu.sync_copy(cnt_hbm.at[r], cv1)
                nch = jnp.max(jnp.where(lane == b, cv1[...], 0))
            base = (b * NWORK + r) * CAP

            for s in range(2):
                cnt_s = jnp.clip(nch - s * HALF, 0, HALF)

                @pl.when(cnt_s > 0)
                def _():
                    if not (r2 == 0 and s == 0):
                        pltpu.sync_copy(bp_hbm.at[pl.ds(base + s * (CAP // 2),
                                                        CAP // 2)], pb)
                        prime(cnt_s)
                    drain(cnt_s)

        plsc.subcore_barrier()
        pltpu.sync_copy(table.at[pl.ds(sid * zr, zr)],
                        out_hbm.at[pl.ds(cid * TH + sid * zr, zr)])

    return _sc_agg


_SC_AGG = {j: _make_sc_agg(j) for j in range(NUM_CLUSTERS)}


def _update_body(j, agg_ref, h_ref, lab_ref, w1_ref, b1_ref, w2_ref, b2_ref, out_ref):
    agg = jnp.concatenate(
        [agg_ref[0:H0, :], agg_ref[TH:TH + (N - H0), :]], axis=0)
    h = h_ref[...]
    z = agg + h
    hid = jnp.maximum(
        jnp.dot(z, w1_ref[...], preferred_element_type=jnp.float32) + b1_ref[...], 0.0)
    new = jnp.dot(hid, w2_ref[...], preferred_element_type=jnp.float32) + b2_ref[...]
    mask = lab_ref[...] == j
    out_ref[...] = jnp.where(mask, new, h)


def _tc_update(j, agg2, h, labels, W1, b1, W2, b2):
    return pl.pallas_call(
        functools.partial(_update_body, j),
        out_shape=jax.ShapeDtypeStruct((N, D), jnp.float32),
    )(agg2, h, labels, W1, b1, W2, b2)


def _pool_body(h_ref, batch_ref, w1_ref, b1_ref, w2_ref, b2_ref, out_ref):
    rows = lax.broadcasted_iota(jnp.int32, (NUM_GRAPHS, N), 0)
    onehot = (rows == batch_ref[...]).astype(jnp.float32)
    pooled = jnp.dot(onehot, h_ref[...], preferred_element_type=jnp.float32)
    hid = jnp.maximum(
        jnp.dot(pooled, w1_ref[...], preferred_element_type=jnp.float32) + b1_ref[...], 0.0)
    out_ref[...] = jnp.dot(hid, w2_ref[...], preferred_element_type=jnp.float32) + b2_ref[...]


def _pool(h, batch_row, W1, b1, W2, b2):
    return pl.pallas_call(
        _pool_body,
        out_shape=jax.ShapeDtypeStruct((NUM_GRAPHS, D), jnp.float32),
    )(h, batch_row, W1, b1, W2, b2)


def kernel(x, conv_W1, conv_b1, conv_W2, conv_b2,
           pool_W1, pool_b1, pool_W2, pool_b2,
           cluster_labels, edge_index, batch):
    src = edge_index[0].astype(jnp.int32)
    dst = edge_index[1].astype(jnp.int32)
    lab1d = cluster_labels.astype(jnp.int32)
    labels = lab1d.reshape(N, 1)
    batch_row = batch.astype(jnp.int32).reshape(1, N)

    bpacked, cnts = _sc_part(src, dst, lab1d)

    h = x
    for i in range(NUM_LAYERS):
        for j in range(NUM_CLUSTERS):
            idx = i * NUM_CLUSTERS + j
            agg2 = _SC_AGG[j](h, bpacked, cnts)
            h = _tc_update(j, agg2, h, labels,
                           conv_W1[idx], conv_b1[idx].reshape(1, D),
                           conv_W2[idx], conv_b2[idx].reshape(1, D))
    return _pool(h, batch_row, pool_W1, pool_b1.reshape(1, D),
                 pool_W2, pool_b2.reshape(1, D))


# speedup vs baseline: 1.0362x; 1.0362x over previous
import dataclasses
import functools

import jax
import jax.numpy as jnp
from jax import lax
from jax.experimental import pallas as pl
from jax.experimental.pallas import tpu as pltpu
from jax.experimental.pallas import tpu_sc as plsc

N = 10000
E = 320000
D = 128
NUM_LAYERS = 2
NUM_CLUSTERS = 4
NUM_GRAPHS = 16

H0 = 5120
TH = 5248
CHUNK = 128
NWORK = 32
EW = E // NWORK
BLK = 2000
CAPC = 80
CAP = CAPC * CHUNK
NBKT = 2 * NUM_CLUSTERS
NREG = NBKT * NWORK
PSH = 8192

_mesh = plsc.VectorSubcoreMesh(core_axis_name="c", subcore_axis_name="s")

_cp = pltpu.CompilerParams()
if "needs_layout_passes" in pltpu.CompilerParams.__dataclass_fields__:
    _cp = dataclasses.replace(_cp, needs_layout_passes=False)


@functools.partial(
    pl.kernel,
    out_type=(
        jax.ShapeDtypeStruct((NREG * CAP,), jnp.int32),
        jax.ShapeDtypeStruct((NWORK, 16), jnp.int32),
    ),
    mesh=_mesh,
    scratch_types=[
        pltpu.VMEM((BLK,), jnp.int32),
        pltpu.VMEM((BLK,), jnp.int32),
        pltpu.VMEM((N,), jnp.int32),
        pltpu.VMEM((CAP,), jnp.int32),
        pltpu.VMEM((CAP,), jnp.int32),
        pltpu.VMEM((CAP,), jnp.int32),
        pltpu.VMEM((CAP,), jnp.int32),
        pltpu.VMEM((CAP,), jnp.int32),
        pltpu.VMEM((CAP,), jnp.int32),
        pltpu.VMEM((CAP,), jnp.int32),
        pltpu.VMEM((CAP,), jnp.int32),
        pltpu.VMEM((16,), jnp.int32),
    ],
    compiler_params=_cp,
)
def _sc_part(src_hbm, dst_hbm, lab_hbm, bp_hbm, cnt_hbm,
             sv, dv, lab, b0, b1, b2, b3, b4, b5, b6, b7, cv):
    cid = lax.axis_index("c")
    sid = lax.axis_index("s")
    w = sid * 2 + cid
    bb = (b0, b1, b2, b3, b4, b5, b6, b7)
    pltpu.sync_copy(lab_hbm, lab)
    lane = lax.iota(jnp.int32, 16)

    cnt = tuple(jnp.int32(0) for _ in range(NBKT))
    for blk in range(EW // BLK):
        pltpu.sync_copy(src_hbm.at[pl.ds(w * EW + blk * BLK, BLK)], sv)
        pltpu.sync_copy(dst_hbm.at[pl.ds(w * EW + blk * BLK, BLK)], dv)

        def body(v, c8):
            s16 = sv[pl.ds(v * 16, 16)]
            d16 = dv[pl.ds(v * 16, 16)]
            k16 = plsc.load_gather(lab, [d16])
            h16 = (d16 >= H0).astype(jnp.int32)
            b16 = k16 * 2 + h16
            p16 = s16 * PSH + (d16 - h16 * H0)
            out = []
            for b in range(NBKT):
                m = b16 == b
                plsc.store_compressed(bb[b].at[pl.ds(c8[b], 16)], p16, mask=m)
                nb_ = jnp.max(plsc.all_reduce_population_count(m))
                out.append(c8[b] + nb_)
            return tuple(out)

        cnt = lax.fori_loop(0, BLK // 16, body, cnt)

    cvec = jnp.zeros((16,), jnp.int32)
    for b in range(NBKT):
        cb = cnt[b]
        for t in range(8):
            off = cb + t * 16
            pad = lane + (t * 16 + w * 128)
            bb[b][pl.ds(off, 16)] = (pad % N) * PSH + H0 + pad % (TH - H0)
        nch = (cb + CHUNK) // CHUNK
        cvec = jnp.where(lane == b, jnp.broadcast_to(nch, (16,)), cvec)
        base = (b * NWORK + w) * CAP
        nblk = (cb + CHUNK + 1023) // 1024

        @pl.loop(0, nblk)
        def _(q):
            pltpu.sync_copy(bb[b].at[pl.ds(q * 1024, 1024)],
                            bp_hbm.at[pl.ds(base + q * 1024, 1024)])

    cv[...] = cvec
    pltpu.sync_copy(cv, cnt_hbm.at[w])


def _make_sc_agg(j):
    @functools.partial(
        pl.kernel,
        out_type=jax.ShapeDtypeStruct((2 * TH, D), jnp.float32),
        mesh=_mesh,
        scratch_types=[
            pltpu.VMEM((CAP // 2,), jnp.int32),
            pltpu.VMEM((3, CHUNK), jnp.int32),
            pltpu.VMEM((3, CHUNK), jnp.int32),
            pltpu.VMEM((CHUNK, D), jnp.float32),
            pltpu.VMEM((CHUNK, D), jnp.float32),
            pltpu.VMEM((CHUNK, D), jnp.float32),
            pltpu.VMEM((CHUNK, D), jnp.float32),
            pltpu.VMEM((16,), jnp.int32),
            pltpu.VMEM((16,), jnp.int32),
            pltpu.VMEM_SHARED((TH, D), jnp.float32),
            pltpu.SemaphoreType.DMA,
            pltpu.SemaphoreType.DMA,
            pltpu.SemaphoreType.DMA,
        ],
        compiler_params=_cp,
    )
    def _sc_agg(h_hbm, bp_hbm, cnt_hbm, out_hbm,
                pb, si2, di2, rows0, rows1, rows2, zbuf, cv0, cv1, table,
                sem0, sem1, sem2):
        cid = lax.axis_index("c")
        sid = lax.axis_index("s")
        HALF = CAPC // 2
        b = j * 2 + cid
        lane = lax.iota(jnp.int32, 16)
        zr = TH // 16

        pltpu.sync_copy(cnt_hbm.at[sid * 2], cv0)
        nch0 = jnp.max(jnp.where(lane == b, cv0[...], 0))
        cnt00 = jnp.clip(nch0, 0, HALF)
        base0 = (b * NWORK + sid * 2) * CAP

        def unpack(k, slot):
            for t in range(CHUNK // 16):
                pv = pb[pl.ds(k * CHUNK + t * 16, 16)]
                si2.at[slot][pl.ds(t * 16, 16)] = lax.shift_right_logical(
                    pv, jnp.int32(13))
                di2.at[slot][pl.ds(t * 16, 16)] = lax.bitwise_and(
                    pv, jnp.int32(PSH - 1))

        ROWS = (rows0, rows1, rows2)
        SEMS = (sem0, sem1, sem2)

        def prime(cnt_s):
            for u in range(3):
                @pl.when(cnt_s > u)
                def _(u=u):
                    unpack(u, u)
                    pltpu.async_copy(h_hbm.at[si2.at[u]], ROWS[u], SEMS[u])

        def drain(cnt_s):
            @pl.loop(0, cnt_s // 3)
            def _(kk):
                k0 = kk * 3
                for u in range(3):
                    pltpu.make_async_copy(h_hbm.at[si2.at[u]], ROWS[u],
                                          SEMS[u]).wait()
                    pltpu.sync_copy(ROWS[u], table.at[di2.at[u]], add=True)

                    @pl.when(k0 + u + 3 < cnt_s)
                    def _(u=u):
                        unpack(k0 + u + 3, u)
                        pltpu.async_copy(h_hbm.at[si2.at[u]], ROWS[u], SEMS[u])

            rem = cnt_s % 3
            for u in range(2):
                @pl.when(rem > u)
                def _(u=u):
                    pltpu.make_async_copy(h_hbm.at[si2.at[u]], ROWS[u],
                                          SEMS[u]).wait()
                    pltpu.sync_copy(ROWS[u], table.at[di2.at[u]], add=True)

        @pl.when(cnt00 > 0)
        def _():
            pltpu.sync_copy(bp_hbm.at[pl.ds(base0, CAP // 2)], pb)
            prime(cnt00)

        z16 = jnp.zeros((16,), jnp.float32)

        @pl.loop(0, CHUNK)
        def _(r):
            for t in range(D // 16):
                zbuf[r, pl.ds(t * 16, 16)] = z16

        pltpu.sync_copy(zbuf, table.at[pl.ds(sid * zr, CHUNK)])
        pltpu.sync_copy(zbuf, table.at[pl.ds(sid * zr + CHUNK, CHUNK)])
        pltpu.sync_copy(zbuf.at[pl.ds(0, zr - 2 * CHUNK)],
                        table.at[pl.ds(sid * zr + 2 * CHUNK, zr - 2 * CHUNK)])
        plsc.subcore_barrier()

        for r2 in range(2):
            r = sid * 2 + r2
            if r2 == 0:
                nch = nch0
            else:
                pltpu.sync_copy(cnt_hbm.at[r], cv1)
                nch = jnp.max(jnp.where(lane == b, cv1[...], 0))
            base = (b * NWORK + r) * CAP

            for s in range(2):
                cnt_s = jnp.clip(nch - s * HALF, 0, HALF)

                @pl.when(cnt_s > 0)
                def _():
                    if not (r2 == 0 and s == 0):
                        pltpu.sync_copy(bp_hbm.at[pl.ds(base + s * (CAP // 2),
                                                        CAP // 2)], pb)
                        prime(cnt_s)
                    drain(cnt_s)

        plsc.subcore_barrier()
        pltpu.sync_copy(table.at[pl.ds(sid * zr, zr)],
                        out_hbm.at[pl.ds(cid * TH + sid * zr, zr)])

    return _sc_agg


_SC_AGG = {j: _make_sc_agg(j) for j in range(NUM_CLUSTERS)}


def _update_body(j, agg_ref, h_ref, lab_ref, w1_ref, b1_ref, w2_ref, b2_ref, out_ref):
    agg = jnp.concatenate(
        [agg_ref[0:H0, :], agg_ref[TH:TH + (N - H0), :]], axis=0)
    h = h_ref[...]
    z = agg + h
    hid = jnp.maximum(
        jnp.dot(z, w1_ref[...], preferred_element_type=jnp.float32) + b1_ref[...], 0.0)
    new = jnp.dot(hid, w2_ref[...], preferred_element_type=jnp.float32) + b2_ref[...]
    mask = lab_ref[...] == j
    out_ref[...] = jnp.where(mask, new, h)


def _tc_update(j, agg2, h, labels, W1, b1, W2, b2):
    return pl.pallas_call(
        functools.partial(_update_body, j),
        out_shape=jax.ShapeDtypeStruct((N, D), jnp.float32),
    )(agg2, h, labels, W1, b1, W2, b2)


def _pool_body(h_ref, batch_ref, w1_ref, b1_ref, w2_ref, b2_ref, out_ref):
    rows = lax.broadcasted_iota(jnp.int32, (NUM_GRAPHS, N), 0)
    onehot = (rows == batch_ref[...]).astype(jnp.float32)
    pooled = jnp.dot(onehot, h_ref[...], preferred_element_type=jnp.float32)
    hid = jnp.maximum(
        jnp.dot(pooled, w1_ref[...], preferred_element_type=jnp.float32) + b1_ref[...], 0.0)
    out_ref[...] = jnp.dot(hid, w2_ref[...], preferred_element_type=jnp.float32) + b2_ref[...]


def _pool(h, batch_row, W1, b1, W2, b2):
    return pl.pallas_call(
        _pool_body,
        out_shape=jax.ShapeDtypeStruct((NUM_GRAPHS, D), jnp.float32),
    )(h, batch_row, W1, b1, W2, b2)


def kernel(x, conv_W1, conv_b1, conv_W2, conv_b2,
           pool_W1, pool_b1, pool_W2, pool_b2,
           cluster_labels, edge_index, batch):
    src = edge_index[0].astype(jnp.int32)
    dst = edge_index[1].astype(jnp.int32)
    lab1d = cluster_labels.astype(jnp.int32)
    labels = lab1d.reshape(N, 1)
    batch_row = batch.astype(jnp.int32).reshape(1, N)

    bpacked, cnts = _sc_part(src, dst, lab1d)

    h = x
    for i in range(NUM_LAYERS):
        for j in range(NUM_CLUSTERS):
            idx = i * NUM_CLUSTERS + j
            agg2 = _SC_AGG[j](h, bpacked, cnts)
            h = _tc_update(j, agg2, h, labels,
                           conv_W1[idx], conv_b1[idx].reshape(1, D),
                           conv_W2[idx], conv_b2[idx].reshape(1, D))
    return _pool(h, batch_row, pool_W1, pool_b1.reshape(1, D),
                 pool_W2, pool_b2.reshape(1, D))
